# baseline (speedup 1.0000x reference)
import functools
import math

import jax
import jax.numpy as jnp
from jax import lax
from jax.experimental import pallas as pl
from jax.experimental.pallas import tpu as pltpu
from jax.experimental.pallas import tpu_sc as plsc

L = 16
NC = 2
NS = 16
NW = NC * NS
TAN2 = math.tan(math.radians(10.0)) ** 2
DIST2 = 25.0


def _make_sc_kernel(n_pad: int, g: int):
    rows_w = n_pad // NW
    chunks_w = rows_w // L
    mesh = plsc.VectorSubcoreMesh(core_axis_name="c", subcore_axis_name="s",
                                  num_cores=NC, num_subcores=NS)

    @functools.partial(
        pl.kernel,
        out_type=jax.ShapeDtypeStruct((n_pad,), jnp.int32),
        mesh=mesh,
        compiler_params=pltpu.CompilerParams(needs_layout_passes=False),
        scratch_types=[
            pltpu.VMEM((rows_w,), jnp.float32),
            pltpu.VMEM((rows_w,), jnp.float32),
            pltpu.VMEM((rows_w,), jnp.float32),
            pltpu.VMEM((rows_w,), jnp.float32),
            pltpu.VMEM((g,), jnp.float32),
            pltpu.VMEM((g,), jnp.float32),
            pltpu.VMEM((g,), jnp.float32),
            pltpu.VMEM((g,), jnp.float32),
            pltpu.VMEM((g,), jnp.float32),
            pltpu.VMEM((g,), jnp.float32),
            pltpu.VMEM((g,), jnp.float32),
            pltpu.VMEM((g,), jnp.float32),
            pltpu.VMEM((rows_w,), jnp.int32),
        ],
    )
    def sc_kernel(p1x_h, p1y_h, p2x_h, p2y_h, gax_h, gay_h, gbx_h, gby_h,
                  out_h, p1x_v, p1y_v, p2x_v, p2y_v, gax_v, gay_v, gbx_v,
                  gby_v, abx_v, aby_v, c_v, d_v, lab_v):
        wid = lax.axis_index("s") * NC + lax.axis_index("c")
        base = wid * rows_w
        pltpu.sync_copy(p1x_h.at[pl.ds(base, rows_w)], p1x_v)
        pltpu.sync_copy(p1y_h.at[pl.ds(base, rows_w)], p1y_v)
        pltpu.sync_copy(p2x_h.at[pl.ds(base, rows_w)], p2x_v)
        pltpu.sync_copy(p2y_h.at[pl.ds(base, rows_w)], p2y_v)
        pltpu.sync_copy(gax_h, gax_v)
        pltpu.sync_copy(gay_h, gay_v)
        pltpu.sync_copy(gbx_h, gbx_v)
        pltpu.sync_copy(gby_h, gby_v)

        for t in range(g // L):
            sl = pl.ds(t * L, L)
            ax = gax_v[sl]
            ay = gay_v[sl]
            abx = gbx_v[sl] - ax
            aby = gby_v[sl] - ay
            abx_v[sl] = abx
            aby_v[sl] = aby
            c_v[sl] = abx * ay - aby * ax
            d_v[sl] = DIST2 * (abx * abx + aby * aby)

        t2 = jnp.float32(TAN2)
        one = jnp.ones((L,), jnp.int32)
        zero = jnp.zeros((L,), jnp.int32)

        def chunk_body(k, carry):
            sls = [pl.ds((k * 2 + i) * L, L) for i in range(2)]
            rows = []
            for sl in sls:
                p1x = p1x_v[sl]
                p1y = p1y_v[sl]
                rows.append((p1x, p1y, p2x_v[sl] - p1x, p2y_v[sl] - p1y))

            def gt_body(j, carry):
                jv, accs = carry
                abx = plsc.load_gather(abx_v, [jv])
                aby = plsc.load_gather(aby_v, [jv])
                c = plsc.load_gather(c_v, [jv])
                d = plsc.load_gather(d_v, [jv])
                new_accs = []
                for (p1x, p1y, d1x, d1y), acc in zip(rows, accs):
                    cross1 = abx * p1y - aby * p1x - c
                    cross_a = d1x * aby - d1y * abx
                    cross2 = cross1 - cross_a
                    dot_a = d1x * abx + d1y * aby
                    m = ((cross1 * cross1 <= d)
                         & (cross2 * cross2 <= d)
                         & (cross_a * cross_a < t2 * (dot_a * dot_a)))
                    new_accs.append(jnp.where(m, one, acc))
                return jv + 1, tuple(new_accs)

            init = (zero, tuple(zero for _ in range(2)))
            _, accs = lax.fori_loop(0, g, gt_body, init, unroll=8)
            for sl, acc in zip(sls, accs):
                lab_v[sl] = acc
            return carry

        lax.fori_loop(0, chunks_w // 2, chunk_body, 0)
        pltpu.sync_copy(lab_v, out_h.at[pl.ds(base, rows_w)])

    return sc_kernel


BR = 512
SC_ROWS = 0


GC = 8


def _tc_body(g, abx_ref, aby_ref, tabx_ref, taby_ref, c_ref, d_ref, pt_ref,
             out_ref):
    p1x = pt_ref[0:1, :]
    p1y = pt_ref[1:2, :]
    d1x = pt_ref[2:3, :] - p1x
    d1y = pt_ref[3:4, :] - p1y

    def gc_body(i, acc):
        s = pl.ds(i * GC, GC)
        abx = abx_ref[s, :]
        aby = aby_ref[s, :]
        tabx = tabx_ref[s, :]
        taby = taby_ref[s, :]
        c = c_ref[s, :]
        d = d_ref[s, :]
        cross1 = abx * p1y - aby * p1x - c
        cross_a = d1x * aby - d1y * abx
        cross2 = cross1 - cross_a
        da = d1x * tabx + d1y * taby
        q1 = cross1 * cross1 - d
        q2 = cross2 * cross2 - d
        q3 = cross_a * cross_a - da * da
        return jnp.minimum(acc, jnp.maximum(jnp.maximum(q1, q2), q3))

    acc = lax.fori_loop(0, g // GC, gc_body,
                        jnp.full((GC, pt_ref.shape[1]), 1.0, jnp.float32),
                        unroll=4)
    out_ref[0, 0, :] = (jnp.min(acc, axis=0) <= 0.0).astype(jnp.int32)


def _tc_labels(pt_pad, gt):
    n_pad = pt_pad.shape[1]
    g = gt.shape[0]
    abx = (gt[:, 2] - gt[:, 0])[:, None]
    aby = (gt[:, 3] - gt[:, 1])[:, None]
    c = abx * gt[:, 1:2] - aby * gt[:, 0:1]
    d = DIST2 * (abx * abx + aby * aby)
    tan = jnp.float32(math.tan(math.radians(10.0)))
    grid = n_pad // BR
    gspec = pl.BlockSpec((g, 1), lambda i: (0, 0))
    out = pl.pallas_call(
        functools.partial(_tc_body, g),
        grid=(grid,),
        in_specs=[gspec, gspec, gspec, gspec, gspec, gspec,
                  pl.BlockSpec((4, BR), lambda i: (0, i))],
        out_specs=pl.BlockSpec((1, 1, BR), lambda i: (i, 0, 0)),
        out_shape=jax.ShapeDtypeStruct((grid, 1, BR), jnp.int32),
    )(abx, aby, tan * abx, tan * aby, c, d, pt_pad)
    return out.reshape(n_pad)


def kernel(proposals, gt_lines):
    n = proposals.shape[0]
    g = gt_lines.shape[0]
    g_pad = -(-g // L) * L
    far = jnp.array([1e6, 1e6, 1e6 + 64.0, 1e6], jnp.float32)
    gt = jnp.concatenate(
        [gt_lines, jnp.broadcast_to(far, (g_pad - g, 4))], axis=0)

    sc_rows = min(SC_ROWS, n - n % (NW * L * 2))
    parts = []
    if sc_rows:
        p_sc = proposals[:sc_rows]
        sc_kernel = _make_sc_kernel(sc_rows, g_pad)
        parts.append(sc_kernel(p_sc[:, 0], p_sc[:, 1], p_sc[:, 2],
                               p_sc[:, 3], gt[:, 0], gt[:, 1], gt[:, 2],
                               gt[:, 3]))
    tc_n = n - sc_rows
    if tc_n:
        tc_pad = -(-tc_n // BR) * BR
        pt_pad = jnp.pad(proposals[sc_rows:].T, ((0, 0), (0, tc_pad - tc_n)))
        parts.append(_tc_labels(pt_pad, gt)[:tc_n])
    labels = jnp.concatenate(parts) if len(parts) > 1 else parts[0]
    return proposals, labels[:n]


# speedup vs baseline: 1.2898x; 1.2898x over previous
import functools
import math

import jax
import jax.numpy as jnp
from jax import lax
from jax.experimental import pallas as pl
from jax.experimental.pallas import tpu as pltpu
from jax.experimental.pallas import tpu_sc as plsc

L = 16
NC = 2
NS = 16
NW = NC * NS
TAN2 = math.tan(math.radians(10.0)) ** 2
DIST2 = 25.0


def _make_sc_kernel(n_pad: int, g: int):
    rows_w = n_pad // NW
    chunks_w = rows_w // L
    mesh = plsc.VectorSubcoreMesh(core_axis_name="c", subcore_axis_name="s",
                                  num_cores=NC, num_subcores=NS)

    @functools.partial(
        pl.kernel,
        out_type=jax.ShapeDtypeStruct((n_pad,), jnp.int32),
        mesh=mesh,
        compiler_params=pltpu.CompilerParams(needs_layout_passes=False),
        scratch_types=[
            pltpu.VMEM((rows_w,), jnp.float32),
            pltpu.VMEM((rows_w,), jnp.float32),
            pltpu.VMEM((rows_w,), jnp.float32),
            pltpu.VMEM((rows_w,), jnp.float32),
            pltpu.VMEM((g,), jnp.float32),
            pltpu.VMEM((g,), jnp.float32),
            pltpu.VMEM((g,), jnp.float32),
            pltpu.VMEM((g,), jnp.float32),
            pltpu.VMEM((g,), jnp.float32),
            pltpu.VMEM((g,), jnp.float32),
            pltpu.VMEM((g,), jnp.float32),
            pltpu.VMEM((g,), jnp.float32),
            pltpu.VMEM((rows_w,), jnp.int32),
        ],
    )
    def sc_kernel(p1x_h, p1y_h, p2x_h, p2y_h, gax_h, gay_h, gbx_h, gby_h,
                  out_h, p1x_v, p1y_v, p2x_v, p2y_v, gax_v, gay_v, gbx_v,
                  gby_v, abx_v, aby_v, c_v, d_v, lab_v):
        wid = lax.axis_index("s") * NC + lax.axis_index("c")
        base = wid * rows_w
        pltpu.sync_copy(p1x_h.at[pl.ds(base, rows_w)], p1x_v)
        pltpu.sync_copy(p1y_h.at[pl.ds(base, rows_w)], p1y_v)
        pltpu.sync_copy(p2x_h.at[pl.ds(base, rows_w)], p2x_v)
        pltpu.sync_copy(p2y_h.at[pl.ds(base, rows_w)], p2y_v)
        pltpu.sync_copy(gax_h, gax_v)
        pltpu.sync_copy(gay_h, gay_v)
        pltpu.sync_copy(gbx_h, gbx_v)
        pltpu.sync_copy(gby_h, gby_v)

        for t in range(g // L):
            sl = pl.ds(t * L, L)
            ax = gax_v[sl]
            ay = gay_v[sl]
            abx = gbx_v[sl] - ax
            aby = gby_v[sl] - ay
            abx_v[sl] = abx
            aby_v[sl] = aby
            c_v[sl] = abx * ay - aby * ax
            d_v[sl] = DIST2 * (abx * abx + aby * aby)

        t2 = jnp.float32(TAN2)
        one = jnp.ones((L,), jnp.int32)
        zero = jnp.zeros((L,), jnp.int32)

        def chunk_body(k, carry):
            sls = [pl.ds((k * 2 + i) * L, L) for i in range(2)]
            rows = []
            for sl in sls:
                p1x = p1x_v[sl]
                p1y = p1y_v[sl]
                rows.append((p1x, p1y, p2x_v[sl] - p1x, p2y_v[sl] - p1y))

            def gt_body(j, carry):
                jv, accs = carry
                abx = plsc.load_gather(abx_v, [jv])
                aby = plsc.load_gather(aby_v, [jv])
                c = plsc.load_gather(c_v, [jv])
                d = plsc.load_gather(d_v, [jv])
                new_accs = []
                for (p1x, p1y, d1x, d1y), acc in zip(rows, accs):
                    cross1 = abx * p1y - aby * p1x - c
                    cross_a = d1x * aby - d1y * abx
                    cross2 = cross1 - cross_a
                    dot_a = d1x * abx + d1y * aby
                    m = ((cross1 * cross1 <= d)
                         & (cross2 * cross2 <= d)
                         & (cross_a * cross_a < t2 * (dot_a * dot_a)))
                    new_accs.append(jnp.where(m, one, acc))
                return jv + 1, tuple(new_accs)

            init = (zero, tuple(zero for _ in range(2)))
            _, accs = lax.fori_loop(0, g, gt_body, init, unroll=8)
            for sl, acc in zip(sls, accs):
                lab_v[sl] = acc
            return carry

        lax.fori_loop(0, chunks_w // 2, chunk_body, 0)
        pltpu.sync_copy(lab_v, out_h.at[pl.ds(base, rows_w)])

    return sc_kernel


BR = 2048
SC_ROWS = 0


GC = 8


def _tc_body(g, abx_ref, aby_ref, tabx_ref, taby_ref, c_ref, d_ref, pt_ref,
             out_ref):
    p1x = pt_ref[0:1, :]
    p1y = pt_ref[1:2, :]
    d1x = pt_ref[2:3, :] - p1x
    d1y = pt_ref[3:4, :] - p1y

    def gc_body(i, acc):
        s = pl.ds(i * GC, GC)
        abx = abx_ref[s, :]
        aby = aby_ref[s, :]
        tabx = tabx_ref[s, :]
        taby = taby_ref[s, :]
        c = c_ref[s, :]
        d = d_ref[s, :]
        cross1 = abx * p1y - aby * p1x - c
        cross_a = d1x * aby - d1y * abx
        cross2 = cross1 - cross_a
        da = d1x * tabx + d1y * taby
        q1 = cross1 * cross1 - d
        q2 = cross2 * cross2 - d
        q3 = cross_a * cross_a - da * da
        return jnp.minimum(acc, jnp.maximum(jnp.maximum(q1, q2), q3))

    acc = lax.fori_loop(0, g // GC, gc_body,
                        jnp.full((GC, pt_ref.shape[1]), 1.0, jnp.float32),
                        unroll=2)
    out_ref[0, 0, :] = (jnp.min(acc, axis=0) <= 0.0).astype(jnp.int32)


def _tc_labels(pt_pad, gt):
    n_pad = pt_pad.shape[1]
    g = gt.shape[0]
    abx = (gt[:, 2] - gt[:, 0])[:, None]
    aby = (gt[:, 3] - gt[:, 1])[:, None]
    c = abx * gt[:, 1:2] - aby * gt[:, 0:1]
    d = DIST2 * (abx * abx + aby * aby)
    tan = jnp.float32(math.tan(math.radians(10.0)))
    grid = n_pad // BR
    gspec = pl.BlockSpec((g, 1), lambda i: (0, 0))
    out = pl.pallas_call(
        functools.partial(_tc_body, g),
        grid=(grid,),
        in_specs=[gspec, gspec, gspec, gspec, gspec, gspec,
                  pl.BlockSpec((4, BR), lambda i: (0, i))],
        out_specs=pl.BlockSpec((1, 1, BR), lambda i: (i, 0, 0)),
        out_shape=jax.ShapeDtypeStruct((grid, 1, BR), jnp.int32),
    )(abx, aby, tan * abx, tan * aby, c, d, pt_pad)
    return out.reshape(n_pad)


def kernel(proposals, gt_lines):
    n = proposals.shape[0]
    g = gt_lines.shape[0]
    g_pad = -(-g // L) * L
    far = jnp.array([1e6, 1e6, 1e6 + 64.0, 1e6], jnp.float32)
    gt = jnp.concatenate(
        [gt_lines, jnp.broadcast_to(far, (g_pad - g, 4))], axis=0)

    sc_rows = min(SC_ROWS, n - n % (NW * L * 2))
    parts = []
    if sc_rows:
        p_sc = proposals[:sc_rows]
        sc_kernel = _make_sc_kernel(sc_rows, g_pad)
        parts.append(sc_kernel(p_sc[:, 0], p_sc[:, 1], p_sc[:, 2],
                               p_sc[:, 3], gt[:, 0], gt[:, 1], gt[:, 2],
                               gt[:, 3]))
    tc_n = n - sc_rows
    if tc_n:
        tc_pad = -(-tc_n // BR) * BR
        pt_pad = jnp.pad(proposals[sc_rows:].T, ((0, 0), (0, tc_pad - tc_n)))
        parts.append(_tc_labels(pt_pad, gt)[:tc_n])
    labels = jnp.concatenate(parts) if len(parts) > 1 else parts[0]
    return proposals, labels[:n]


# speedup vs baseline: 1.5412x; 1.1949x over previous
import functools
import math

import jax
import jax.numpy as jnp
from jax import lax
from jax.experimental import pallas as pl
from jax.experimental.pallas import tpu as pltpu
from jax.experimental.pallas import tpu_sc as plsc

L = 16
NC = 2
NS = 16
NW = NC * NS
TAN2 = math.tan(math.radians(10.0)) ** 2
DIST2 = 25.0


def _make_sc_kernel(n_pad: int, g: int):
    rows_w = n_pad // NW
    chunks_w = rows_w // L
    mesh = plsc.VectorSubcoreMesh(core_axis_name="c", subcore_axis_name="s",
                                  num_cores=NC, num_subcores=NS)

    @functools.partial(
        pl.kernel,
        out_type=jax.ShapeDtypeStruct((n_pad,), jnp.int32),
        mesh=mesh,
        compiler_params=pltpu.CompilerParams(needs_layout_passes=False),
        scratch_types=[
            pltpu.VMEM((rows_w,), jnp.float32),
            pltpu.VMEM((rows_w,), jnp.float32),
            pltpu.VMEM((rows_w,), jnp.float32),
            pltpu.VMEM((rows_w,), jnp.float32),
            pltpu.VMEM((g,), jnp.float32),
            pltpu.VMEM((g,), jnp.float32),
            pltpu.VMEM((g,), jnp.float32),
            pltpu.VMEM((g,), jnp.float32),
            pltpu.VMEM((g,), jnp.float32),
            pltpu.VMEM((g,), jnp.float32),
            pltpu.VMEM((g,), jnp.float32),
            pltpu.VMEM((g,), jnp.float32),
            pltpu.VMEM((rows_w,), jnp.int32),
        ],
    )
    def sc_kernel(p1x_h, p1y_h, p2x_h, p2y_h, gax_h, gay_h, gbx_h, gby_h,
                  out_h, p1x_v, p1y_v, p2x_v, p2y_v, gax_v, gay_v, gbx_v,
                  gby_v, abx_v, aby_v, c_v, d_v, lab_v):
        wid = lax.axis_index("s") * NC + lax.axis_index("c")
        base = wid * rows_w
        pltpu.sync_copy(p1x_h.at[pl.ds(base, rows_w)], p1x_v)
        pltpu.sync_copy(p1y_h.at[pl.ds(base, rows_w)], p1y_v)
        pltpu.sync_copy(p2x_h.at[pl.ds(base, rows_w)], p2x_v)
        pltpu.sync_copy(p2y_h.at[pl.ds(base, rows_w)], p2y_v)
        pltpu.sync_copy(gax_h, gax_v)
        pltpu.sync_copy(gay_h, gay_v)
        pltpu.sync_copy(gbx_h, gbx_v)
        pltpu.sync_copy(gby_h, gby_v)

        for t in range(g // L):
            sl = pl.ds(t * L, L)
            ax = gax_v[sl]
            ay = gay_v[sl]
            abx = gbx_v[sl] - ax
            aby = gby_v[sl] - ay
            abx_v[sl] = abx
            aby_v[sl] = aby
            c_v[sl] = abx * ay - aby * ax
            d_v[sl] = DIST2 * (abx * abx + aby * aby)

        t2 = jnp.float32(TAN2)
        one = jnp.ones((L,), jnp.int32)
        zero = jnp.zeros((L,), jnp.int32)

        def chunk_body(k, carry):
            sls = [pl.ds((k * 2 + i) * L, L) for i in range(2)]
            rows = []
            for sl in sls:
                p1x = p1x_v[sl]
                p1y = p1y_v[sl]
                rows.append((p1x, p1y, p2x_v[sl] - p1x, p2y_v[sl] - p1y))

            def gt_body(j, carry):
                jv, accs = carry
                abx = plsc.load_gather(abx_v, [jv])
                aby = plsc.load_gather(aby_v, [jv])
                c = plsc.load_gather(c_v, [jv])
                d = plsc.load_gather(d_v, [jv])
                new_accs = []
                for (p1x, p1y, d1x, d1y), acc in zip(rows, accs):
                    cross1 = abx * p1y - aby * p1x - c
                    cross_a = d1x * aby - d1y * abx
                    cross2 = cross1 - cross_a
                    dot_a = d1x * abx + d1y * aby
                    m = ((cross1 * cross1 <= d)
                         & (cross2 * cross2 <= d)
                         & (cross_a * cross_a < t2 * (dot_a * dot_a)))
                    new_accs.append(jnp.where(m, one, acc))
                return jv + 1, tuple(new_accs)

            init = (zero, tuple(zero for _ in range(2)))
            _, accs = lax.fori_loop(0, g, gt_body, init, unroll=8)
            for sl, acc in zip(sls, accs):
                lab_v[sl] = acc
            return carry

        lax.fori_loop(0, chunks_w // 2, chunk_body, 0)
        pltpu.sync_copy(lab_v, out_h.at[pl.ds(base, rows_w)])

    return sc_kernel


BR = 2048
SC_ROWS = 0


GC = 8


def _tc_body(g, abx_ref, aby_ref, tabx_ref, taby_ref, c_ref, d_ref, pt_ref,
             out_ref):
    p1x = pt_ref[0:1, :]
    p1y = pt_ref[1:2, :]
    d1x = pt_ref[2:3, :] - p1x
    d1y = pt_ref[3:4, :] - p1y

    def gc_body(i, acc):
        s = pl.ds(i * GC, GC)
        abx = abx_ref[s, :]
        aby = aby_ref[s, :]
        tabx = tabx_ref[s, :]
        taby = taby_ref[s, :]
        c = c_ref[s, :]
        d = d_ref[s, :]
        cross1 = abx * p1y - aby * p1x - c
        cross_a = d1x * aby - d1y * abx
        cross2 = cross1 - cross_a
        da = d1x * tabx + d1y * taby
        q1 = cross1 * cross1 - d
        q2 = cross2 * cross2 - d
        q3 = cross_a * cross_a - da * da
        return jnp.minimum(acc, jnp.maximum(jnp.maximum(q1, q2), q3))

    acc = lax.fori_loop(0, g // GC, gc_body,
                        jnp.full((GC, pt_ref.shape[1]), 1.0, jnp.float32),
                        unroll=4)
    out_ref[0, 0, :] = (jnp.min(acc, axis=0) <= 0.0).astype(jnp.int32)


def _tc_labels(pt_pad, gt):
    n_pad = pt_pad.shape[1]
    g = gt.shape[0]
    abx = (gt[:, 2] - gt[:, 0])[:, None]
    aby = (gt[:, 3] - gt[:, 1])[:, None]
    c = abx * gt[:, 1:2] - aby * gt[:, 0:1]
    d = DIST2 * (abx * abx + aby * aby)
    tan = jnp.float32(math.tan(math.radians(10.0)))
    grid = n_pad // BR
    gspec = pl.BlockSpec((g, 1), lambda i: (0, 0))
    out = pl.pallas_call(
        functools.partial(_tc_body, g),
        grid=(grid,),
        in_specs=[gspec, gspec, gspec, gspec, gspec, gspec,
                  pl.BlockSpec((4, BR), lambda i: (0, i))],
        out_specs=pl.BlockSpec((1, 1, BR), lambda i: (i, 0, 0)),
        out_shape=jax.ShapeDtypeStruct((grid, 1, BR), jnp.int32),
    )(abx, aby, tan * abx, tan * aby, c, d, pt_pad)
    return out.reshape(n_pad)


def kernel(proposals, gt_lines):
    n = proposals.shape[0]
    g = gt_lines.shape[0]
    g_pad = -(-g // L) * L
    far = jnp.array([1e6, 1e6, 1e6 + 64.0, 1e6], jnp.float32)
    gt = jnp.concatenate(
        [gt_lines, jnp.broadcast_to(far, (g_pad - g, 4))], axis=0)

    sc_rows = min(SC_ROWS, n - n % (NW * L * 2))
    parts = []
    if sc_rows:
        p_sc = proposals[:sc_rows]
        sc_kernel = _make_sc_kernel(sc_rows, g_pad)
        parts.append(sc_kernel(p_sc[:, 0], p_sc[:, 1], p_sc[:, 2],
                               p_sc[:, 3], gt[:, 0], gt[:, 1], gt[:, 2],
                               gt[:, 3]))
    tc_n = n - sc_rows
    if tc_n:
        tc_pad = -(-tc_n // BR) * BR
        pt_pad = jnp.pad(proposals[sc_rows:].T, ((0, 0), (0, tc_pad - tc_n)))
        parts.append(_tc_labels(pt_pad, gt)[:tc_n])
    labels = jnp.concatenate(parts) if len(parts) > 1 else parts[0]
    return proposals, labels[:n]


# speedup vs baseline: 1.7543x; 1.1383x over previous
import functools
import math

import jax
import jax.numpy as jnp
from jax import lax
from jax.experimental import pallas as pl
from jax.experimental.pallas import tpu as pltpu
from jax.experimental.pallas import tpu_sc as plsc

L = 16
NC = 2
NS = 16
NW = NC * NS
TAN2 = math.tan(math.radians(10.0)) ** 2
DIST2 = 25.0


def _make_sc_kernel(n_pad: int, g: int):
    rows_w = n_pad // NW
    chunks_w = rows_w // L
    mesh = plsc.VectorSubcoreMesh(core_axis_name="c", subcore_axis_name="s",
                                  num_cores=NC, num_subcores=NS)

    @functools.partial(
        pl.kernel,
        out_type=jax.ShapeDtypeStruct((n_pad,), jnp.int32),
        mesh=mesh,
        compiler_params=pltpu.CompilerParams(needs_layout_passes=False),
        scratch_types=[
            pltpu.VMEM((rows_w,), jnp.float32),
            pltpu.VMEM((rows_w,), jnp.float32),
            pltpu.VMEM((rows_w,), jnp.float32),
            pltpu.VMEM((rows_w,), jnp.float32),
            pltpu.VMEM((g,), jnp.float32),
            pltpu.VMEM((g,), jnp.float32),
            pltpu.VMEM((g,), jnp.float32),
            pltpu.VMEM((g,), jnp.float32),
            pltpu.VMEM((g,), jnp.float32),
            pltpu.VMEM((g,), jnp.float32),
            pltpu.VMEM((g,), jnp.float32),
            pltpu.VMEM((g,), jnp.float32),
            pltpu.VMEM((rows_w,), jnp.int32),
        ],
    )
    def sc_kernel(p1x_h, p1y_h, p2x_h, p2y_h, gax_h, gay_h, gbx_h, gby_h,
                  out_h, p1x_v, p1y_v, p2x_v, p2y_v, gax_v, gay_v, gbx_v,
                  gby_v, abx_v, aby_v, c_v, d_v, lab_v):
        wid = lax.axis_index("s") * NC + lax.axis_index("c")
        base = wid * rows_w
        pltpu.sync_copy(p1x_h.at[pl.ds(base, rows_w)], p1x_v)
        pltpu.sync_copy(p1y_h.at[pl.ds(base, rows_w)], p1y_v)
        pltpu.sync_copy(p2x_h.at[pl.ds(base, rows_w)], p2x_v)
        pltpu.sync_copy(p2y_h.at[pl.ds(base, rows_w)], p2y_v)
        pltpu.sync_copy(gax_h, gax_v)
        pltpu.sync_copy(gay_h, gay_v)
        pltpu.sync_copy(gbx_h, gbx_v)
        pltpu.sync_copy(gby_h, gby_v)

        for t in range(g // L):
            sl = pl.ds(t * L, L)
            ax = gax_v[sl]
            ay = gay_v[sl]
            abx = gbx_v[sl] - ax
            aby = gby_v[sl] - ay
            abx_v[sl] = abx
            aby_v[sl] = aby
            c_v[sl] = abx * ay - aby * ax
            d_v[sl] = DIST2 * (abx * abx + aby * aby)

        t2 = jnp.float32(TAN2)
        one = jnp.ones((L,), jnp.int32)
        zero = jnp.zeros((L,), jnp.int32)

        def chunk_body(k, carry):
            sls = [pl.ds((k * 2 + i) * L, L) for i in range(2)]
            rows = []
            for sl in sls:
                p1x = p1x_v[sl]
                p1y = p1y_v[sl]
                rows.append((p1x, p1y, p2x_v[sl] - p1x, p2y_v[sl] - p1y))

            def gt_body(j, carry):
                jv, accs = carry
                abx = plsc.load_gather(abx_v, [jv])
                aby = plsc.load_gather(aby_v, [jv])
                c = plsc.load_gather(c_v, [jv])
                d = plsc.load_gather(d_v, [jv])
                new_accs = []
                for (p1x, p1y, d1x, d1y), acc in zip(rows, accs):
                    cross1 = abx * p1y - aby * p1x - c
                    cross_a = d1x * aby - d1y * abx
                    cross2 = cross1 - cross_a
                    dot_a = d1x * abx + d1y * aby
                    m = ((cross1 * cross1 <= d)
                         & (cross2 * cross2 <= d)
                         & (cross_a * cross_a < t2 * (dot_a * dot_a)))
                    new_accs.append(jnp.where(m, one, acc))
                return jv + 1, tuple(new_accs)

            init = (zero, tuple(zero for _ in range(2)))
            _, accs = lax.fori_loop(0, g, gt_body, init, unroll=8)
            for sl, acc in zip(sls, accs):
                lab_v[sl] = acc
            return carry

        lax.fori_loop(0, chunks_w // 2, chunk_body, 0)
        pltpu.sync_copy(lab_v, out_h.at[pl.ds(base, rows_w)])

    return sc_kernel


BR = 2048
SC_ROWS = 0


GC = 8


def _tc_body(g, abx_ref, aby_ref, tabx_ref, taby_ref, c_ref, d_ref, pt_ref,
             out_ref):
    p1x = pt_ref[0:1, :]
    p1y = pt_ref[1:2, :]
    d1x = pt_ref[2:3, :] - p1x
    d1y = pt_ref[3:4, :] - p1y

    def gc_body(i, acc):
        s = pl.ds(i * GC, GC)
        abx = abx_ref[s, :]
        aby = aby_ref[s, :]
        tabx = tabx_ref[s, :]
        taby = taby_ref[s, :]
        c = c_ref[s, :]
        d = d_ref[s, :]
        cross1 = abx * p1y - aby * p1x - c
        cross_a = d1x * aby - d1y * abx
        cross2 = cross1 - cross_a
        da = d1x * tabx + d1y * taby
        q1 = cross1 * cross1 - d
        q2 = cross2 * cross2 - d
        q3 = cross_a * cross_a - da * da
        return jnp.minimum(acc, jnp.maximum(jnp.maximum(q1, q2), q3))

    acc = lax.fori_loop(0, g // GC, gc_body,
                        jnp.full((GC, pt_ref.shape[1]), 1.0, jnp.float32),
                        unroll=8)
    out_ref[0, 0, :] = (jnp.min(acc, axis=0) <= 0.0).astype(jnp.int32)


def _tc_labels(pt_pad, gt):
    n_pad = pt_pad.shape[1]
    g = gt.shape[0]
    abx = (gt[:, 2] - gt[:, 0])[:, None]
    aby = (gt[:, 3] - gt[:, 1])[:, None]
    c = abx * gt[:, 1:2] - aby * gt[:, 0:1]
    d = DIST2 * (abx * abx + aby * aby)
    tan = jnp.float32(math.tan(math.radians(10.0)))
    grid = n_pad // BR
    gspec = pl.BlockSpec((g, 1), lambda i: (0, 0))
    out = pl.pallas_call(
        functools.partial(_tc_body, g),
        grid=(grid,),
        in_specs=[gspec, gspec, gspec, gspec, gspec, gspec,
                  pl.BlockSpec((4, BR), lambda i: (0, i))],
        out_specs=pl.BlockSpec((1, 1, BR), lambda i: (i, 0, 0)),
        out_shape=jax.ShapeDtypeStruct((grid, 1, BR), jnp.int32),
    )(abx, aby, tan * abx, tan * aby, c, d, pt_pad)
    return out.reshape(n_pad)


def kernel(proposals, gt_lines):
    n = proposals.shape[0]
    g = gt_lines.shape[0]
    g_pad = -(-g // L) * L
    far = jnp.array([1e6, 1e6, 1e6 + 64.0, 1e6], jnp.float32)
    gt = jnp.concatenate(
        [gt_lines, jnp.broadcast_to(far, (g_pad - g, 4))], axis=0)

    sc_rows = min(SC_ROWS, n - n % (NW * L * 2))
    parts = []
    if sc_rows:
        p_sc = proposals[:sc_rows]
        sc_kernel = _make_sc_kernel(sc_rows, g_pad)
        parts.append(sc_kernel(p_sc[:, 0], p_sc[:, 1], p_sc[:, 2],
                               p_sc[:, 3], gt[:, 0], gt[:, 1], gt[:, 2],
                               gt[:, 3]))
    tc_n = n - sc_rows
    if tc_n:
        tc_pad = -(-tc_n // BR) * BR
        pt_pad = jnp.pad(proposals[sc_rows:].T, ((0, 0), (0, tc_pad - tc_n)))
        parts.append(_tc_labels(pt_pad, gt)[:tc_n])
    labels = jnp.concatenate(parts) if len(parts) > 1 else parts[0]
    return proposals, labels[:n]


# speedup vs baseline: 1.8667x; 1.0640x over previous
import functools
import math

import jax
import jax.numpy as jnp
from jax import lax
from jax.experimental import pallas as pl
from jax.experimental.pallas import tpu as pltpu
from jax.experimental.pallas import tpu_sc as plsc

L = 16
NC = 2
NS = 16
NW = NC * NS
TAN2 = math.tan(math.radians(10.0)) ** 2
DIST2 = 25.0


def _make_sc_kernel(n_pad: int, g: int):
    rows_w = n_pad // NW
    chunks_w = rows_w // L
    mesh = plsc.VectorSubcoreMesh(core_axis_name="c", subcore_axis_name="s",
                                  num_cores=NC, num_subcores=NS)

    @functools.partial(
        pl.kernel,
        out_type=jax.ShapeDtypeStruct((n_pad,), jnp.int32),
        mesh=mesh,
        compiler_params=pltpu.CompilerParams(needs_layout_passes=False),
        scratch_types=[
            pltpu.VMEM((rows_w,), jnp.float32),
            pltpu.VMEM((rows_w,), jnp.float32),
            pltpu.VMEM((rows_w,), jnp.float32),
            pltpu.VMEM((rows_w,), jnp.float32),
            pltpu.VMEM((g,), jnp.float32),
            pltpu.VMEM((g,), jnp.float32),
            pltpu.VMEM((g,), jnp.float32),
            pltpu.VMEM((g,), jnp.float32),
            pltpu.VMEM((g,), jnp.float32),
            pltpu.VMEM((g,), jnp.float32),
            pltpu.VMEM((g,), jnp.float32),
            pltpu.VMEM((g,), jnp.float32),
            pltpu.VMEM((rows_w,), jnp.int32),
        ],
    )
    def sc_kernel(p1x_h, p1y_h, p2x_h, p2y_h, gax_h, gay_h, gbx_h, gby_h,
                  out_h, p1x_v, p1y_v, p2x_v, p2y_v, gax_v, gay_v, gbx_v,
                  gby_v, abx_v, aby_v, c_v, d_v, lab_v):
        wid = lax.axis_index("s") * NC + lax.axis_index("c")
        base = wid * rows_w
        pltpu.sync_copy(p1x_h.at[pl.ds(base, rows_w)], p1x_v)
        pltpu.sync_copy(p1y_h.at[pl.ds(base, rows_w)], p1y_v)
        pltpu.sync_copy(p2x_h.at[pl.ds(base, rows_w)], p2x_v)
        pltpu.sync_copy(p2y_h.at[pl.ds(base, rows_w)], p2y_v)
        pltpu.sync_copy(gax_h, gax_v)
        pltpu.sync_copy(gay_h, gay_v)
        pltpu.sync_copy(gbx_h, gbx_v)
        pltpu.sync_copy(gby_h, gby_v)

        for t in range(g // L):
            sl = pl.ds(t * L, L)
            ax = gax_v[sl]
            ay = gay_v[sl]
            abx = gbx_v[sl] - ax
            aby = gby_v[sl] - ay
            abx_v[sl] = abx
            aby_v[sl] = aby
            c_v[sl] = abx * ay - aby * ax
            d_v[sl] = DIST2 * (abx * abx + aby * aby)

        t2 = jnp.float32(TAN2)
        one = jnp.ones((L,), jnp.int32)
        zero = jnp.zeros((L,), jnp.int32)

        def chunk_body(k, carry):
            sls = [pl.ds((k * 2 + i) * L, L) for i in range(2)]
            rows = []
            for sl in sls:
                p1x = p1x_v[sl]
                p1y = p1y_v[sl]
                rows.append((p1x, p1y, p2x_v[sl] - p1x, p2y_v[sl] - p1y))

            def gt_body(j, carry):
                jv, accs = carry
                abx = plsc.load_gather(abx_v, [jv])
                aby = plsc.load_gather(aby_v, [jv])
                c = plsc.load_gather(c_v, [jv])
                d = plsc.load_gather(d_v, [jv])
                new_accs = []
                for (p1x, p1y, d1x, d1y), acc in zip(rows, accs):
                    cross1 = abx * p1y - aby * p1x - c
                    cross_a = d1x * aby - d1y * abx
                    cross2 = cross1 - cross_a
                    dot_a = d1x * abx + d1y * aby
                    m = ((cross1 * cross1 <= d)
                         & (cross2 * cross2 <= d)
                         & (cross_a * cross_a < t2 * (dot_a * dot_a)))
                    new_accs.append(jnp.where(m, one, acc))
                return jv + 1, tuple(new_accs)

            init = (zero, tuple(zero for _ in range(2)))
            _, accs = lax.fori_loop(0, g, gt_body, init, unroll=8)
            for sl, acc in zip(sls, accs):
                lab_v[sl] = acc
            return carry

        lax.fori_loop(0, chunks_w // 2, chunk_body, 0)
        pltpu.sync_copy(lab_v, out_h.at[pl.ds(base, rows_w)])

    return sc_kernel


BR = 2048
SC_ROWS = 0


GC = 8


def _tc_body(g, abx_ref, aby_ref, tabx_ref, taby_ref, c_ref, d_ref, pt_ref,
             out_ref):
    p1x = pt_ref[0:1, :]
    p1y = pt_ref[1:2, :]
    d1x = pt_ref[2:3, :] - p1x
    d1y = pt_ref[3:4, :] - p1y

    def gc_body(i, acc):
        s = slice(i * GC, (i + 1) * GC)
        abx = abx_ref[s, :]
        aby = aby_ref[s, :]
        tabx = tabx_ref[s, :]
        taby = taby_ref[s, :]
        c = c_ref[s, :]
        d = d_ref[s, :]
        cross1 = abx * p1y - aby * p1x - c
        cross_a = d1x * aby - d1y * abx
        cross2 = cross1 - cross_a
        da = d1x * tabx + d1y * taby
        q1 = cross1 * cross1 - d
        q2 = cross2 * cross2 - d
        q3 = cross_a * cross_a - da * da
        return jnp.minimum(acc, jnp.maximum(jnp.maximum(q1, q2), q3))

    acc = jnp.full((GC, pt_ref.shape[1]), 1.0, jnp.float32)
    for i in range(g // GC):
        acc = gc_body(i, acc)
    out_ref[0, 0, :] = (jnp.min(acc, axis=0) <= 0.0).astype(jnp.int32)


def _tc_labels(pt_pad, gt):
    n_pad = pt_pad.shape[1]
    g = gt.shape[0]
    abx = (gt[:, 2] - gt[:, 0])[:, None]
    aby = (gt[:, 3] - gt[:, 1])[:, None]
    c = abx * gt[:, 1:2] - aby * gt[:, 0:1]
    d = DIST2 * (abx * abx + aby * aby)
    tan = jnp.float32(math.tan(math.radians(10.0)))
    grid = n_pad // BR
    gspec = pl.BlockSpec((g, 1), lambda i: (0, 0))
    out = pl.pallas_call(
        functools.partial(_tc_body, g),
        grid=(grid,),
        in_specs=[gspec, gspec, gspec, gspec, gspec, gspec,
                  pl.BlockSpec((4, BR), lambda i: (0, i))],
        out_specs=pl.BlockSpec((1, 1, BR), lambda i: (i, 0, 0)),
        out_shape=jax.ShapeDtypeStruct((grid, 1, BR), jnp.int32),
    )(abx, aby, tan * abx, tan * aby, c, d, pt_pad)
    return out.reshape(n_pad)


def kernel(proposals, gt_lines):
    n = proposals.shape[0]
    g = gt_lines.shape[0]
    g_pad = -(-g // L) * L
    far = jnp.array([1e6, 1e6, 1e6 + 64.0, 1e6], jnp.float32)
    gt = jnp.concatenate(
        [gt_lines, jnp.broadcast_to(far, (g_pad - g, 4))], axis=0)

    sc_rows = min(SC_ROWS, n - n % (NW * L * 2))
    parts = []
    if sc_rows:
        p_sc = proposals[:sc_rows]
        sc_kernel = _make_sc_kernel(sc_rows, g_pad)
        parts.append(sc_kernel(p_sc[:, 0], p_sc[:, 1], p_sc[:, 2],
                               p_sc[:, 3], gt[:, 0], gt[:, 1], gt[:, 2],
                               gt[:, 3]))
    tc_n = n - sc_rows
    if tc_n:
        tc_pad = -(-tc_n // BR) * BR
        pt_pad = jnp.pad(proposals[sc_rows:].T, ((0, 0), (0, tc_pad - tc_n)))
        parts.append(_tc_labels(pt_pad, gt)[:tc_n])
    labels = jnp.concatenate(parts) if len(parts) > 1 else parts[0]
    return proposals, labels[:n]


# speedup vs baseline: 1.8675x; 1.0004x over previous
import functools
import math

import jax
import jax.numpy as jnp
from jax.experimental import pallas as pl

TAN = math.tan(math.radians(10.0))
DIST2 = 25.0
BR = 2048
GC = 8


def _tc_body(g, abx_ref, aby_ref, tabx_ref, taby_ref, c_ref, d_ref, pt_ref,
             out_ref):
    p1x = pt_ref[0:1, :]
    p1y = pt_ref[1:2, :]
    d1x = pt_ref[2:3, :] - p1x
    d1y = pt_ref[3:4, :] - p1y

    acc = jnp.full((GC, pt_ref.shape[1]), 1.0, jnp.float32)
    for i in range(g // GC):
        s = slice(i * GC, (i + 1) * GC)
        abx = abx_ref[s, :]
        aby = aby_ref[s, :]
        tabx = tabx_ref[s, :]
        taby = taby_ref[s, :]
        c = c_ref[s, :]
        d = d_ref[s, :]
        cross1 = abx * p1y - aby * p1x - c
        cross_a = d1x * aby - d1y * abx
        cross2 = cross1 - cross_a
        da = d1x * tabx + d1y * taby
        q1 = cross1 * cross1 - d
        q2 = cross2 * cross2 - d
        q3 = cross_a * cross_a - da * da
        acc = jnp.minimum(acc, jnp.maximum(jnp.maximum(q1, q2), q3))

    out_ref[0, 0, :] = (jnp.min(acc, axis=0) <= 0.0).astype(jnp.int32)


def _tc_labels(pt_pad, gt):
    n_pad = pt_pad.shape[1]
    g = gt.shape[0]
    abx = (gt[:, 2] - gt[:, 0])[:, None]
    aby = (gt[:, 3] - gt[:, 1])[:, None]
    c = abx * gt[:, 1:2] - aby * gt[:, 0:1]
    d = DIST2 * (abx * abx + aby * aby)
    tan = jnp.float32(TAN)
    grid = n_pad // BR
    gspec = pl.BlockSpec((g, 1), lambda i: (0, 0))
    out = pl.pallas_call(
        functools.partial(_tc_body, g),
        grid=(grid,),
        in_specs=[gspec, gspec, gspec, gspec, gspec, gspec,
                  pl.BlockSpec((4, BR), lambda i: (0, i))],
        out_specs=pl.BlockSpec((1, 1, BR), lambda i: (i, 0, 0)),
        out_shape=jax.ShapeDtypeStruct((grid, 1, BR), jnp.int32),
    )(abx, aby, tan * abx, tan * aby, c, d, pt_pad)
    return out.reshape(n_pad)


def kernel(proposals, gt_lines):
    n = proposals.shape[0]
    g = gt_lines.shape[0]
    g_pad = -(-g // 16) * 16
    far = jnp.array([1e6, 1e6, 1e6 + 64.0, 1e6], jnp.float32)
    gt = jnp.concatenate(
        [gt_lines, jnp.broadcast_to(far, (g_pad - g, 4))], axis=0)
    n_pad = -(-n // BR) * BR
    pt_pad = jnp.pad(proposals.T, ((0, 0), (0, n_pad - n)))
    labels = _tc_labels(pt_pad, gt)[:n]
    return proposals, labels
